# Initial kernel scaffold; baseline (speedup 1.0000x reference)
import jax
import jax.numpy as jnp
from jax.experimental import pallas as pl


def kernel(input, emb, W1, b1, W2, b2):
    raise NotImplementedError("write your pallas kernel here")


# speedup vs baseline: 3.1583x; 3.1583x over previous
import functools

import jax
import jax.numpy as jnp
from jax import lax
from jax.experimental import pallas as pl
from jax.experimental.pallas import tpu as pltpu
from jax.experimental.pallas import tpu_sc as plsc

VOCAB = 100000
EMB = 128
DPAD = 16
ROWS = 2000
BTOK = 1024 * 200

def _table_body(emb_ref, w1_ref, b1_ref, w2_ref, b2_ref, out_ref):
    h = jnp.dot(emb_ref[...], w1_ref[...], preferred_element_type=jnp.float32)
    h = jnp.maximum(h + b1_ref[...], 0.0)
    y = jnp.dot(h, w2_ref[...], preferred_element_type=jnp.float32)
    out_ref[...] = (y + b2_ref[...]).astype(jnp.int32)


_table_call = pl.pallas_call(
    _table_body,
    grid=(VOCAB // ROWS,),
    in_specs=[
        pl.BlockSpec((ROWS, EMB), lambda i: (i, 0)),
        pl.BlockSpec((EMB, 2 * EMB), lambda i: (0, 0)),
        pl.BlockSpec((1, 2 * EMB), lambda i: (0, 0)),
        pl.BlockSpec((2 * EMB, DPAD), lambda i: (0, 0)),
        pl.BlockSpec((1, DPAD), lambda i: (0, 0)),
    ],
    out_specs=pl.BlockSpec((ROWS, DPAD), lambda i: (i, 0)),
    out_shape=jax.ShapeDtypeStruct((VOCAB, DPAD), jnp.int32),
)


@functools.cache
def _gather_call():
    info = plsc.get_sparse_core_info()
    nc = info.num_cores
    bpw = BTOK // (nc * info.num_subcores)

    def body(table_hbm, idx_hbm, out_hbm, idx_v, rows_v, sem):
        wid = lax.axis_index("s") * nc + lax.axis_index("c")
        base = wid * bpw
        pltpu.sync_copy(idx_hbm.at[pl.ds(base, bpw)], idx_v)
        pltpu.async_copy(table_hbm.at[idx_v], rows_v, sem).wait()
        pltpu.sync_copy(rows_v, out_hbm.at[pl.ds(base, bpw)])

    return pl.kernel(
        body,
        out_type=jax.ShapeDtypeStruct((BTOK, DPAD), jnp.int32),
        mesh=plsc.VectorSubcoreMesh(core_axis_name="c", subcore_axis_name="s"),
        scratch_types=[
            pltpu.VMEM((bpw,), jnp.int32),
            pltpu.VMEM((bpw, DPAD), jnp.int32),
            pltpu.SemaphoreType.DMA,
        ],
        compiler_params=pltpu.CompilerParams(use_tc_tiling_on_sc=False),
    )


def kernel(input, emb, W1, b1, W2, b2):
    w2p = jnp.zeros((2 * EMB, DPAD), W2.dtype).at[:, :2].set(W2)
    b2p = jnp.zeros((1, DPAD), b2.dtype).at[0, :2].set(b2)
    table = _table_call(emb, W1, b1.reshape(1, -1), w2p, b2p)
    idx = input.reshape(-1).astype(jnp.int32)
    rows = _gather_call()(table, idx)
    out = rows.reshape(input.shape[0], input.shape[1], DPAD)[..., :2]
    return out.astype(jnp.int64)


# speedup vs baseline: 3.5295x; 1.1175x over previous
import functools

import jax
import jax.numpy as jnp
from jax import lax
from jax.experimental import pallas as pl
from jax.experimental.pallas import tpu as pltpu
from jax.experimental.pallas import tpu_sc as plsc

VOCAB = 100000
EMB = 128
DPAD = 16
ROWS = 2000
BTOK = 1024 * 200
LANES = 16


def _table_body(emb_ref, w1_ref, b1_ref, w2_ref, b2_ref, out_ref):
    h = jnp.dot(emb_ref[...], w1_ref[...], preferred_element_type=jnp.float32)
    h = jnp.maximum(h + b1_ref[...], 0.0)
    y = jnp.dot(h, w2_ref[...], preferred_element_type=jnp.float32)
    out_ref[...] = (y + b2_ref[...]).astype(jnp.int32)


_table_call = pl.pallas_call(
    _table_body,
    grid=(VOCAB // ROWS,),
    in_specs=[
        pl.BlockSpec((ROWS, EMB), lambda i: (i, 0)),
        pl.BlockSpec((EMB, 2 * EMB), lambda i: (0, 0)),
        pl.BlockSpec((1, 2 * EMB), lambda i: (0, 0)),
        pl.BlockSpec((2 * EMB, DPAD), lambda i: (0, 0)),
        pl.BlockSpec((1, DPAD), lambda i: (0, 0)),
    ],
    out_specs=pl.BlockSpec((ROWS, DPAD), lambda i: (i, 0)),
    out_shape=jax.ShapeDtypeStruct((VOCAB, DPAD), jnp.int32),
)


@functools.cache
def _gather_call():
    info = plsc.get_sparse_core_info()
    nc = info.num_cores
    bpw = BTOK // (nc * info.num_subcores)

    def body(table_hbm, idx_hbm, out_hbm, idx_v, rows_v, sem):
        wid = lax.axis_index("s") * nc + lax.axis_index("c")
        base = wid * bpw
        pltpu.sync_copy(idx_hbm.at[pl.ds(base, bpw)], idx_v)
        pltpu.async_copy(table_hbm.at[idx_v], rows_v, sem).wait()
        pltpu.sync_copy(rows_v, out_hbm.at[pl.ds(base, bpw)])

    return pl.kernel(
        body,
        out_type=jax.ShapeDtypeStruct((BTOK, DPAD), jnp.int32),
        mesh=plsc.VectorSubcoreMesh(core_axis_name="c", subcore_axis_name="s"),
        scratch_types=[
            pltpu.VMEM((bpw,), jnp.int32),
            pltpu.VMEM((bpw, DPAD), jnp.int32),
            pltpu.SemaphoreType.DMA,
        ],
        compiler_params=pltpu.CompilerParams(use_tc_tiling_on_sc=False),
    )


def kernel(input, emb, W1, b1, W2, b2):
    w2p = jnp.zeros((2 * EMB, DPAD), W2.dtype).at[:, :2].set(W2)
    b2p = jnp.zeros((1, DPAD), b2.dtype).at[0, :2].set(b2)
    table = _table_call(emb, W1, b1.reshape(1, -1), w2p, b2p)
    idx = input.T.reshape(-1).astype(jnp.int32)
    rows = _gather_call()(table, idx)
    flat = rows.reshape(-1)
    shp = (input.shape[1], input.shape[0])
    lo = flat[0::DPAD].reshape(shp)
    hi = flat[1::DPAD].reshape(shp)
    out = jnp.stack([lo, hi], axis=-1).transpose(1, 0, 2)
    return out.astype(jnp.int64)


# speedup vs baseline: 3.9696x; 1.1247x over previous
import functools

import jax
import jax.numpy as jnp
from jax import lax
from jax.experimental import pallas as pl
from jax.experimental.pallas import tpu as pltpu
from jax.experimental.pallas import tpu_sc as plsc

VOCAB = 100000
EMB = 128
DPAD = 16
ROWS = 10000
BTOK = 1024 * 200
LANES = 16


def _table_body(emb_ref, w1_ref, b1_ref, w2_ref, b2_ref, out_ref):
    h = jnp.dot(emb_ref[...], w1_ref[...], preferred_element_type=jnp.float32)
    h = jnp.maximum(h + b1_ref[...], 0.0)
    y = jnp.dot(h, w2_ref[...], preferred_element_type=jnp.float32)
    out_ref[...] = (y + b2_ref[...]).astype(jnp.int32)


_table_call = pl.pallas_call(
    _table_body,
    grid=(VOCAB // ROWS,),
    in_specs=[
        pl.BlockSpec((ROWS, EMB), lambda i: (i, 0)),
        pl.BlockSpec((EMB, 2 * EMB), lambda i: (0, 0)),
        pl.BlockSpec((1, 2 * EMB), lambda i: (0, 0)),
        pl.BlockSpec((2 * EMB, DPAD), lambda i: (0, 0)),
        pl.BlockSpec((1, DPAD), lambda i: (0, 0)),
    ],
    out_specs=pl.BlockSpec((ROWS, DPAD), lambda i: (i, 0)),
    out_shape=jax.ShapeDtypeStruct((VOCAB, DPAD), jnp.int32),
)


@functools.cache
def _gather_call():
    info = plsc.get_sparse_core_info()
    nc = info.num_cores
    bpw = BTOK // (nc * info.num_subcores)

    def body(table_hbm, idx_hbm, out_hbm, idx_v, rows_v, sem):
        wid = lax.axis_index("s") * nc + lax.axis_index("c")
        base = wid * bpw
        pltpu.sync_copy(idx_hbm.at[pl.ds(base, bpw)], idx_v)
        pltpu.async_copy(table_hbm.at[idx_v], rows_v, sem).wait()
        pltpu.sync_copy(rows_v, out_hbm.at[pl.ds(base, bpw)])

    return pl.kernel(
        body,
        out_type=jax.ShapeDtypeStruct((BTOK, DPAD), jnp.int32),
        mesh=plsc.VectorSubcoreMesh(core_axis_name="c", subcore_axis_name="s"),
        scratch_types=[
            pltpu.VMEM((bpw,), jnp.int32),
            pltpu.VMEM((bpw, DPAD), jnp.int32),
            pltpu.SemaphoreType.DMA,
        ],
        compiler_params=pltpu.CompilerParams(use_tc_tiling_on_sc=False),
    )


def kernel(input, emb, W1, b1, W2, b2):
    w2p = jnp.zeros((2 * EMB, DPAD), W2.dtype).at[:, :2].set(W2)
    b2p = jnp.zeros((1, DPAD), b2.dtype).at[0, :2].set(b2)
    table = _table_call(emb, W1, b1.reshape(1, -1), w2p, b2p)
    idx = input.T.reshape(-1).astype(jnp.int32)
    rows = _gather_call()(table, idx)
    flat = rows.reshape(-1)
    shp = (input.shape[1], input.shape[0])
    lo = flat[0::DPAD].reshape(shp)
    hi = flat[1::DPAD].reshape(shp)
    out = jnp.stack([lo, hi], axis=-1).transpose(1, 0, 2)
    return out.astype(jnp.int64)
